# baseline (speedup 1.0000x reference)
import functools

import jax
import jax.numpy as jnp
from jax import lax
from jax.experimental import pallas as pl
from jax.experimental.pallas import tpu as pltpu
from jax.experimental.pallas import tpu_sc as plsc

B = 1024
E = 64
V = 100000

_NC = 2
_NS = 16
_NW = _NC * _NS
_BPW = B // _NW

@functools.cache
def _make_sc_gather():
    mesh = plsc.VectorSubcoreMesh(core_axis_name="c", subcore_axis_name="s")

    @functools.partial(
        pl.kernel,
        mesh=mesh,
        out_type=jax.ShapeDtypeStruct((B, E), jnp.float32),
        scratch_types=[
            pltpu.VMEM((_BPW,), jnp.int32),
            pltpu.VMEM((_BPW, E), jnp.float32),
            pltpu.SemaphoreType.DMA,
        ],
        compiler_params=pltpu.CompilerParams(use_tc_tiling_on_sc=False),
    )
    def _sc_gather(emb_hbm, idx_hbm, out_hbm, idx_v, rows_v, sem):
        wid = lax.axis_index("s") * _NC + lax.axis_index("c")
        base = wid * _BPW
        pltpu.sync_copy(idx_hbm.at[pl.ds(base, _BPW)], idx_v)
        pltpu.async_copy(emb_hbm.at[idx_v], rows_v, sem).wait()
        pltpu.sync_copy(rows_v, out_hbm.at[pl.ds(base, _BPW)])

    return _sc_gather


_BB = 16


def _proj_body(h_ref, w_ref, b_ref, out_ref):
    acc = lax.dot_general(
        h_ref[...], w_ref[...],
        dimension_numbers=(((1,), (1,)), ((), ())),
        preferred_element_type=jnp.float32,
    )
    out_ref[...] = acc + b_ref[...]


def _tc_project(h, W, b2d):
    grid = (B // _BB,)
    return pl.pallas_call(
        _proj_body,
        grid=grid,
        in_specs=[
            pl.BlockSpec((_BB, E), lambda i: (i, 0)),
            pl.BlockSpec((V, E), lambda i: (0, 0)),
            pl.BlockSpec((1, V), lambda i: (0, 0)),
        ],
        out_specs=pl.BlockSpec((_BB, V), lambda i: (i, 0)),
        out_shape=jax.ShapeDtypeStruct((B, V), jnp.float32),
        compiler_params=pltpu.CompilerParams(
            vmem_limit_bytes=64 * 1024 * 1024,
        ),
    )(h, W, b2d)


def kernel(x, emb, W, b):
    h = _make_sc_gather()(emb, x.astype(jnp.int32))
    return _tc_project(h, W, b.reshape(1, V))


# speedup vs baseline: 2.8437x; 2.8437x over previous
import functools

import jax
import jax.numpy as jnp
from jax import lax
from jax.experimental import pallas as pl
from jax.experimental.pallas import tpu as pltpu
from jax.experimental.pallas import tpu_sc as plsc

B = 1024
E = 64
V = 100000

_NC = 2
_NS = 16
_NW = _NC * _NS
_BPW = B // _NW

@functools.cache
def _make_sc_gather():
    mesh = plsc.VectorSubcoreMesh(core_axis_name="c", subcore_axis_name="s")

    @functools.partial(
        pl.kernel,
        mesh=mesh,
        out_type=jax.ShapeDtypeStruct((B, E), jnp.float32),
        scratch_types=[
            pltpu.VMEM((_BPW,), jnp.int32),
            pltpu.VMEM((_BPW, E), jnp.float32),
            pltpu.SemaphoreType.DMA,
        ],
        compiler_params=pltpu.CompilerParams(use_tc_tiling_on_sc=False),
    )
    def _sc_gather(emb_hbm, idx_hbm, out_hbm, idx_v, rows_v, sem):
        wid = lax.axis_index("s") * _NC + lax.axis_index("c")
        base = wid * _BPW
        pltpu.sync_copy(idx_hbm.at[pl.ds(base, _BPW)], idx_v)
        pltpu.async_copy(emb_hbm.at[idx_v], rows_v, sem).wait()
        pltpu.sync_copy(rows_v, out_hbm.at[pl.ds(base, _BPW)])

    return _sc_gather


_VB = 2000


def _proj_body(w_ref, h_ref, b_ref, out_ref):
    acc = lax.dot_general(
        w_ref[...], h_ref[...],
        dimension_numbers=(((1,), (1,)), ((), ())),
        preferred_element_type=jnp.float32,
    )
    out_ref[...] = acc + b_ref[...]


def _tc_project(h, W, bcol):
    grid = (V // _VB,)
    outT = pl.pallas_call(
        _proj_body,
        grid=grid,
        in_specs=[
            pl.BlockSpec((_VB, E), lambda i: (i, 0)),
            pl.BlockSpec((B, E), lambda i: (0, 0)),
            pl.BlockSpec((_VB, 1), lambda i: (i, 0)),
        ],
        out_specs=pl.BlockSpec((_VB, B), lambda i: (i, 0)),
        out_shape=jax.ShapeDtypeStruct((V, B), jnp.float32),
    )(W, h, bcol)
    return outT.T


def kernel(x, emb, W, b):
    h = _make_sc_gather()(emb, x.astype(jnp.int32))
    return _tc_project(h, W, b.reshape(V, 1))


# speedup vs baseline: 4.2320x; 1.4882x over previous
import functools

import jax
import jax.numpy as jnp
from jax import lax
from jax.experimental import pallas as pl
from jax.experimental.pallas import tpu as pltpu
from jax.experimental.pallas import tpu_sc as plsc

B = 1024
E = 64
V = 100000

_NC = 2
_NS = 16
_NW = _NC * _NS
_BPW = B // _NW

@functools.cache
def _make_sc_gather():
    mesh = plsc.VectorSubcoreMesh(core_axis_name="c", subcore_axis_name="s")

    @functools.partial(
        pl.kernel,
        mesh=mesh,
        out_type=jax.ShapeDtypeStruct((B, E), jnp.float32),
        scratch_types=[
            pltpu.VMEM((_BPW,), jnp.int32),
            pltpu.VMEM((_BPW, E), jnp.float32),
            pltpu.SemaphoreType.DMA,
        ],
        compiler_params=pltpu.CompilerParams(use_tc_tiling_on_sc=False),
    )
    def _sc_gather(emb_hbm, idx_hbm, out_hbm, idx_v, rows_v, sem):
        wid = lax.axis_index("s") * _NC + lax.axis_index("c")
        base = wid * _BPW
        pltpu.sync_copy(idx_hbm.at[pl.ds(base, _BPW)], idx_v)
        pltpu.async_copy(emb_hbm.at[idx_v], rows_v, sem).wait()
        pltpu.sync_copy(rows_v, out_hbm.at[pl.ds(base, _BPW)])

    return _sc_gather


_VB = 2048
_NG = (V + _VB - 1) // _VB
_VPAD = _NG * _VB


def _proj_body(wt_ref, h_ref, b_ref, out_ref):
    acc = lax.dot_general(
        wt_ref[...], h_ref[...],
        dimension_numbers=(((0,), (1,)), ((), ())),
        preferred_element_type=jnp.float32,
    )
    ones_row = jnp.ones((1, B), dtype=jnp.float32)
    bias = lax.dot_general(
        b_ref[0], ones_row,
        dimension_numbers=(((0,), (0,)), ((), ())),
        preferred_element_type=jnp.float32,
    )
    out_ref[...] = acc + bias


def _tc_project(h, Wt, b3):
    outT = pl.pallas_call(
        _proj_body,
        grid=(_NG,),
        in_specs=[
            pl.BlockSpec((E, _VB), lambda i: (0, i)),
            pl.BlockSpec((B, E), lambda i: (0, 0)),
            pl.BlockSpec((1, 1, _VB), lambda i: (i, 0, 0)),
        ],
        out_specs=pl.BlockSpec((_VB, B), lambda i: (i, 0)),
        out_shape=jax.ShapeDtypeStruct((V, B), jnp.float32),
    )(Wt, h, b3)
    return outT.T


def kernel(x, emb, W, b):
    h = _make_sc_gather()(emb, x.astype(jnp.int32))
    Wt = W.T
    b3 = jnp.pad(b, (0, _VPAD - V)).reshape(_NG, 1, _VB)
    return _tc_project(h, Wt, b3)


# speedup vs baseline: 4.5674x; 1.0793x over previous
import functools

import jax
import jax.numpy as jnp
from jax import lax
from jax.experimental import pallas as pl
from jax.experimental.pallas import tpu as pltpu
from jax.experimental.pallas import tpu_sc as plsc

B = 1024
E = 64
V = 100000

_NC = 2
_NS = 16
_NW = _NC * _NS
_BPW = B // _NW

@functools.cache
def _make_sc_gather():
    mesh = plsc.VectorSubcoreMesh(core_axis_name="c", subcore_axis_name="s")

    @functools.partial(
        pl.kernel,
        mesh=mesh,
        out_type=jax.ShapeDtypeStruct((B, E), jnp.float32),
        scratch_types=[
            pltpu.VMEM((_BPW,), jnp.int32),
            pltpu.VMEM((_BPW, E), jnp.float32),
            pltpu.SemaphoreType.DMA,
        ],
        compiler_params=pltpu.CompilerParams(use_tc_tiling_on_sc=False),
    )
    def _sc_gather(emb_hbm, idx_hbm, out_hbm, idx_v, rows_v, sem):
        wid = lax.axis_index("s") * _NC + lax.axis_index("c")
        base = wid * _BPW
        pltpu.sync_copy(idx_hbm.at[pl.ds(base, _BPW)], idx_v)
        pltpu.async_copy(emb_hbm.at[idx_v], rows_v, sem).wait()
        pltpu.sync_copy(rows_v, out_hbm.at[pl.ds(base, _BPW)])

    return _sc_gather


_EPW = E // _NW


@functools.cache
def _make_sc_gather_t():
    mesh = plsc.VectorSubcoreMesh(core_axis_name="c", subcore_axis_name="s")

    @functools.partial(
        pl.kernel,
        mesh=mesh,
        out_type=jax.ShapeDtypeStruct((E, B), jnp.float32),
        scratch_types=[
            pltpu.VMEM((B,), jnp.int32),
            pltpu.VMEM((V,), jnp.float32),
            pltpu.VMEM((B,), jnp.float32),
        ],
        compiler_params=pltpu.CompilerParams(
            use_tc_tiling_on_sc=False, needs_layout_passes=False),
    )
    def _sc_gather_t(embT_hbm, idx_hbm, out_hbm, idx_v, row_v, hrow_v):
        wid = lax.axis_index("s") * _NC + lax.axis_index("c")
        pltpu.sync_copy(idx_hbm, idx_v)
        for r in range(_EPW):
            e = wid * _EPW + r
            pltpu.sync_copy(embT_hbm.at[e], row_v)

            def body(j, carry):
                idx = idx_v[pl.ds(j * 16, 16)]
                hrow_v[pl.ds(j * 16, 16)] = plsc.load_gather(row_v, [idx])
                return carry

            lax.fori_loop(0, B // 16, body, 0)
            pltpu.sync_copy(hrow_v, out_hbm.at[e])

    return _sc_gather_t


_VB = 2048
_NG = (V + _VB - 1) // _VB
_VPAD = _NG * _VB


def _proj_body(wt_ref, h_ref, b_ref, out_ref):
    acc = lax.dot_general(
        wt_ref[...], h_ref[...],
        dimension_numbers=(((0,), (0,)), ((), ())),
        preferred_element_type=jnp.float32,
    )
    ones_row = jnp.ones((1, B), dtype=jnp.float32)
    bias = lax.dot_general(
        b_ref[0], ones_row,
        dimension_numbers=(((0,), (0,)), ((), ())),
        preferred_element_type=jnp.float32,
    )
    out_ref[...] = acc + bias


def _tc_project(h, Wt, b3):
    outT = pl.pallas_call(
        _proj_body,
        grid=(_NG,),
        in_specs=[
            pl.BlockSpec((E, _VB), lambda i: (0, i)),
            pl.BlockSpec((E, B), lambda i: (0, 0)),
            pl.BlockSpec((1, 1, _VB), lambda i: (i, 0, 0)),
        ],
        out_specs=pl.BlockSpec((_VB, B), lambda i: (i, 0)),
        out_shape=jax.ShapeDtypeStruct((V, B), jnp.float32),
    )(Wt, h, b3)
    return outT.T


def kernel(x, emb, W, b):
    hT = _make_sc_gather_t()(emb.T, x.astype(jnp.int32))
    Wt = W.T
    b3 = jnp.pad(b, (0, _VPAD - V)).reshape(_NG, 1, _VB)
    return _tc_project(hT, Wt, b3)


# speedup vs baseline: 5.5555x; 1.2163x over previous
import functools

import jax
import jax.numpy as jnp
from jax import lax
from jax.experimental import pallas as pl
from jax.experimental.pallas import tpu as pltpu
from jax.experimental.pallas import tpu_sc as plsc

B = 1024
E = 64
V = 100000

_NC = 2
_NS = 16
_NW = _NC * _NS
_BPW = B // _NW

@functools.cache
def _make_sc_gather():
    mesh = plsc.VectorSubcoreMesh(core_axis_name="c", subcore_axis_name="s")

    @functools.partial(
        pl.kernel,
        mesh=mesh,
        out_type=jax.ShapeDtypeStruct((B, E), jnp.float32),
        scratch_types=[
            pltpu.VMEM((_BPW,), jnp.int32),
            pltpu.VMEM((_BPW, E), jnp.float32),
            pltpu.SemaphoreType.DMA,
        ],
        compiler_params=pltpu.CompilerParams(use_tc_tiling_on_sc=False),
    )
    def _sc_gather(emb_hbm, idx_hbm, out_hbm, idx_v, rows_v, sem):
        wid = lax.axis_index("s") * _NC + lax.axis_index("c")
        base = wid * _BPW
        pltpu.sync_copy(idx_hbm.at[pl.ds(base, _BPW)], idx_v)
        pltpu.async_copy(emb_hbm.at[idx_v], rows_v, sem).wait()
        pltpu.sync_copy(rows_v, out_hbm.at[pl.ds(base, _BPW)])

    return _sc_gather


_EPW = E // _NW


@functools.cache
def _make_sc_gather_t():
    mesh = plsc.VectorSubcoreMesh(core_axis_name="c", subcore_axis_name="s")

    @functools.partial(
        pl.kernel,
        mesh=mesh,
        out_type=jax.ShapeDtypeStruct((E, B), jnp.float32),
        scratch_types=[
            pltpu.VMEM((B,), jnp.int32),
            pltpu.VMEM((V,), jnp.float32),
            pltpu.VMEM((B,), jnp.float32),
        ],
        compiler_params=pltpu.CompilerParams(
            use_tc_tiling_on_sc=True, needs_layout_passes=False),
    )
    def _sc_gather_t(embT_hbm, idx_hbm, out_hbm, idx_v, row_v, hrow_v):
        wid = lax.axis_index("s") * _NC + lax.axis_index("c")
        pltpu.sync_copy(idx_hbm, idx_v)
        for r in range(_EPW):
            e = wid * _EPW + r
            pltpu.sync_copy(embT_hbm.at[e], row_v)

            def body(j, carry):
                idx = idx_v[pl.ds(j * 16, 16)]
                hrow_v[pl.ds(j * 16, 16)] = plsc.load_gather(row_v, [idx])
                return carry

            lax.fori_loop(0, B // 16, body, 0)
            pltpu.sync_copy(hrow_v, out_hbm.at[e])

    return _sc_gather_t


_VB = 2048
_NG = (V + _VB - 1) // _VB
_VPAD = _NG * _VB


def _proj_body(wt_ref, h_ref, b_ref, out_ref):
    acc = lax.dot_general(
        wt_ref[...], h_ref[...],
        dimension_numbers=(((0,), (0,)), ((), ())),
        preferred_element_type=jnp.float32,
    )
    ones_row = jnp.ones((1, B), dtype=jnp.float32)
    bias = lax.dot_general(
        b_ref[0], ones_row,
        dimension_numbers=(((0,), (0,)), ((), ())),
        preferred_element_type=jnp.float32,
    )
    out_ref[...] = acc + bias


def _tc_project(h, Wt, b3):
    outT = pl.pallas_call(
        _proj_body,
        grid=(_NG,),
        in_specs=[
            pl.BlockSpec((E, _VB), lambda i: (0, i)),
            pl.BlockSpec((E, B), lambda i: (0, 0)),
            pl.BlockSpec((1, 1, _VB), lambda i: (i, 0, 0)),
        ],
        out_specs=pl.BlockSpec((_VB, B), lambda i: (i, 0)),
        out_shape=jax.ShapeDtypeStruct((V, B), jnp.float32),
    )(Wt, h, b3)
    return outT.T


def kernel(x, emb, W, b):
    hT = _make_sc_gather_t()(emb.T, x.astype(jnp.int32))
    Wt = W.T
    b3 = jnp.pad(b, (0, _VPAD - V)).reshape(_NG, 1, _VB)
    return _tc_project(hT, Wt, b3)


# speedup vs baseline: 5.6275x; 1.0129x over previous
import functools

import jax
import jax.numpy as jnp
from jax import lax
from jax.experimental import pallas as pl
from jax.experimental.pallas import tpu as pltpu
from jax.experimental.pallas import tpu_sc as plsc

B = 1024
E = 64
V = 100000

_NC = 2
_NS = 16
_NW = _NC * _NS
_BPW = B // _NW

@functools.cache
def _make_sc_gather():
    mesh = plsc.VectorSubcoreMesh(core_axis_name="c", subcore_axis_name="s")

    @functools.partial(
        pl.kernel,
        mesh=mesh,
        out_type=jax.ShapeDtypeStruct((B, E), jnp.float32),
        scratch_types=[
            pltpu.VMEM((_BPW,), jnp.int32),
            pltpu.VMEM((_BPW, E), jnp.float32),
            pltpu.SemaphoreType.DMA,
        ],
        compiler_params=pltpu.CompilerParams(use_tc_tiling_on_sc=False),
    )
    def _sc_gather(emb_hbm, idx_hbm, out_hbm, idx_v, rows_v, sem):
        wid = lax.axis_index("s") * _NC + lax.axis_index("c")
        base = wid * _BPW
        pltpu.sync_copy(idx_hbm.at[pl.ds(base, _BPW)], idx_v)
        pltpu.async_copy(emb_hbm.at[idx_v], rows_v, sem).wait()
        pltpu.sync_copy(rows_v, out_hbm.at[pl.ds(base, _BPW)])

    return _sc_gather


_EPW = E // _NW


@functools.cache
def _make_sc_gather_t():
    mesh = plsc.VectorSubcoreMesh(core_axis_name="c", subcore_axis_name="s")

    @functools.partial(
        pl.kernel,
        mesh=mesh,
        out_type=jax.ShapeDtypeStruct((E, B), jnp.float32),
        scratch_types=[
            pltpu.VMEM((B,), jnp.int32),
            pltpu.VMEM((V,), jnp.float32),
            pltpu.VMEM((B,), jnp.float32),
        ],
        compiler_params=pltpu.CompilerParams(
            use_tc_tiling_on_sc=True, needs_layout_passes=False),
    )
    def _sc_gather_t(embT_hbm, idx_hbm, out_hbm, idx_v, row_v, hrow_v):
        wid = lax.axis_index("s") * _NC + lax.axis_index("c")
        pltpu.sync_copy(idx_hbm, idx_v)
        for r in range(_EPW):
            e = wid * _EPW + r
            pltpu.sync_copy(embT_hbm.at[e], row_v)

            def body(j, carry):
                idx = idx_v[pl.ds(j * 16, 16)]
                hrow_v[pl.ds(j * 16, 16)] = plsc.load_gather(row_v, [idx])
                return carry

            lax.fori_loop(0, B // 16, body, 0)
            pltpu.sync_copy(hrow_v, out_hbm.at[e])

    return _sc_gather_t


_VB = 4096
_NG = (V + _VB - 1) // _VB
_VPAD = _NG * _VB


def _proj_body(wt_ref, h_ref, b_ref, out_ref):
    acc = lax.dot_general(
        wt_ref[...], h_ref[...],
        dimension_numbers=(((0,), (0,)), ((), ())),
        preferred_element_type=jnp.float32,
    )
    ones_row = jnp.ones((1, B), dtype=jnp.float32)
    bias = lax.dot_general(
        b_ref[0], ones_row,
        dimension_numbers=(((0,), (0,)), ((), ())),
        preferred_element_type=jnp.float32,
    )
    out_ref[...] = acc + bias


def _tc_project(h, Wt, b3):
    outT = pl.pallas_call(
        _proj_body,
        grid=(_NG,),
        in_specs=[
            pl.BlockSpec((E, _VB), lambda i: (0, i)),
            pl.BlockSpec((E, B), lambda i: (0, 0)),
            pl.BlockSpec((1, 1, _VB), lambda i: (i, 0, 0)),
        ],
        out_specs=pl.BlockSpec((_VB, B), lambda i: (i, 0)),
        out_shape=jax.ShapeDtypeStruct((V, B), jnp.float32),
    )(Wt, h, b3)
    return outT.T


def kernel(x, emb, W, b):
    hT = _make_sc_gather_t()(emb.T, x.astype(jnp.int32))
    Wt = W.T
    b3 = jnp.pad(b, (0, _VPAD - V)).reshape(_NG, 1, _VB)
    return _tc_project(hT, Wt, b3)
